# baseline (speedup 1.0000x reference)
import functools

import jax
import jax.numpy as jnp
from jax.experimental import pallas as pl
from jax.experimental.pallas import tpu as pltpu

_IN_F, _HID_F, _OUT_F = 4, 50, 3
_W2_ROW = 16
_B2_ROW = 144
_OUT_W = 8
_HID_W = 56


def _mlp_kernel_t(xt_ref, w1_ref, b1t_ref, w2_ref, o_ref):
    ht = jax.lax.dot_general(
        w1_ref[...], xt_ref[...].astype(jnp.bfloat16),
        dimension_numbers=(((0,), (0,)), ((), ())),
        preferred_element_type=jnp.float32,
    )
    ht = jnp.maximum(ht.astype(jnp.bfloat16) + b1t_ref[...],
                     jnp.bfloat16(0.0))
    yt = jax.lax.dot_general(
        w2_ref[...], ht,
        dimension_numbers=(((0,), (0,)), ((), ())),
        preferred_element_type=jnp.float32,
    )
    o_ref[...] = yt[:_OUT_F, :]


@functools.partial(jax.jit, static_argnames=("tile_n",))
def _forward(x, params_packed, tile_n=65536):
    B = x.shape[0]
    xt = x.T
    tn = min(tile_n, max(128, -(-B // 128) * 128))
    n_pad = -(-B // tn) * tn
    if n_pad != B:
        xt = jnp.pad(xt, ((0, 0), (0, n_pad - B)))

    w1 = params_packed[0:_IN_F, :_HID_W]
    b1 = params_packed[8:9, :_HID_W]
    b2 = params_packed[_B2_ROW:_B2_ROW + 1, :_OUT_W]
    w2 = params_packed[_W2_ROW:_W2_ROW + _HID_W, :_OUT_W]
    b1 = b1.at[0, _HID_F].set(1.0)
    w2 = w2.at[_HID_F, :].set(b2[0, :])
    w1 = w1.astype(jnp.bfloat16)
    w2 = w2.astype(jnp.bfloat16)
    b1t = b1.T.astype(jnp.bfloat16)

    grid = (n_pad // tn,)
    cost = pl.CostEstimate(
        flops=2 * n_pad * (_IN_F * _HID_W + _HID_W * _OUT_W),
        transcendentals=0,
        bytes_accessed=(n_pad * _IN_F + 152 * 128 + n_pad * _OUT_F) * 4,
    )
    out_t = pl.pallas_call(
        _mlp_kernel_t,
        out_shape=jax.ShapeDtypeStruct((_OUT_F, n_pad), jnp.float32),
        grid=grid,
        in_specs=[
            pl.BlockSpec((_IN_F, tn), lambda i: (0, i)),
            pl.BlockSpec((_IN_F, _HID_W), lambda i: (0, 0)),
            pl.BlockSpec((_HID_W, 1), lambda i: (0, 0)),
            pl.BlockSpec((_HID_W, _OUT_W), lambda i: (0, 0)),
        ],
        out_specs=pl.BlockSpec((_OUT_F, tn), lambda i: (0, i)),
        compiler_params=pltpu.CompilerParams(
            dimension_semantics=("parallel",),
        ),
        cost_estimate=cost,
    )(xt, w1, b1t, w2)
    return out_t[:, :B].T


def kernel(x, params_packed):
    return _forward(x, params_packed)


# speedup vs baseline: 1.0196x; 1.0196x over previous
import functools

import jax
import jax.numpy as jnp
from jax.experimental import pallas as pl
from jax.experimental.pallas import tpu as pltpu

_IN_F, _HID_F, _OUT_F = 4, 50, 3
_W2_ROW = 16
_B2_ROW = 144
_OUT_W = 8
_HID_W = 56


def _mlp_kernel_t(xt_ref, w1_ref, b1t_ref, w2_ref, o_ref, h_ref):
    ht = jax.lax.dot_general(
        w1_ref[...], xt_ref[...].astype(jnp.bfloat16),
        dimension_numbers=(((0,), (0,)), ((), ())),
        preferred_element_type=jnp.float32,
    )
    h_ref[...] = jnp.maximum(ht.astype(jnp.bfloat16) + b1t_ref[...],
                             jnp.bfloat16(0.0))
    yt = jax.lax.dot_general(
        w2_ref[...], h_ref[...],
        dimension_numbers=(((0,), (0,)), ((), ())),
        preferred_element_type=jnp.float32,
    )
    o_ref[...] = yt[:_OUT_F, :]


@functools.partial(jax.jit, static_argnames=("tile_n",))
def _forward(x, params_packed, tile_n=131072):
    B = x.shape[0]
    xt = x.T
    tn = min(tile_n, max(128, -(-B // 128) * 128))
    n_pad = -(-B // tn) * tn
    if n_pad != B:
        xt = jnp.pad(xt, ((0, 0), (0, n_pad - B)))

    w1 = params_packed[0:_IN_F, :_HID_W]
    b1 = params_packed[8:9, :_HID_W]
    b2 = params_packed[_B2_ROW:_B2_ROW + 1, :_OUT_W]
    w2 = params_packed[_W2_ROW:_W2_ROW + _HID_W, :_OUT_W]
    b1 = b1.at[0, _HID_F].set(1.0)
    w2 = w2.at[_HID_F, :].set(b2[0, :])
    w1 = w1.astype(jnp.bfloat16)
    w2 = w2.astype(jnp.bfloat16)
    b1t = b1.T.astype(jnp.bfloat16)

    grid = (n_pad // tn,)
    cost = pl.CostEstimate(
        flops=2 * n_pad * (_IN_F * _HID_W + _HID_W * _OUT_W),
        transcendentals=0,
        bytes_accessed=(n_pad * _IN_F + 152 * 128 + n_pad * _OUT_F) * 4,
    )
    out_t = pl.pallas_call(
        _mlp_kernel_t,
        out_shape=jax.ShapeDtypeStruct((_OUT_F, n_pad), jnp.float32),
        grid=grid,
        in_specs=[
            pl.BlockSpec((_IN_F, tn), lambda i: (0, i)),
            pl.BlockSpec((_IN_F, _HID_W), lambda i: (0, 0)),
            pl.BlockSpec((_HID_W, 1), lambda i: (0, 0)),
            pl.BlockSpec((_HID_W, _OUT_W), lambda i: (0, 0)),
        ],
        out_specs=pl.BlockSpec((_OUT_F, tn), lambda i: (0, i)),
        scratch_shapes=[pltpu.VMEM((_HID_W, tn), jnp.bfloat16)],
        compiler_params=pltpu.CompilerParams(
            dimension_semantics=("parallel",),
        ),
        cost_estimate=cost,
    )(xt, w1, b1t, w2)
    return out_t[:, :B].T


def kernel(x, params_packed):
    return _forward(x, params_packed)


# speedup vs baseline: 1.1245x; 1.1029x over previous
import functools

import jax
import jax.numpy as jnp
from jax.experimental import pallas as pl
from jax.experimental.pallas import tpu as pltpu

_IN_F, _HID_F, _OUT_F = 4, 50, 3
_W2_ROW = 16
_B2_ROW = 144
_OUT_W = 8
_HID_W = 56


def _mlp_kernel_t(xa_ref, xb_ref, w1_ref, b1t_ref, w2_ref, o_ref, h_ref):
    xp = jnp.concatenate(
        [xa_ref[...], xb_ref[...]], axis=0).astype(jnp.bfloat16)
    ht = jax.lax.dot_general(
        w1_ref[...], xp,
        dimension_numbers=(((0,), (0,)), ((), ())),
        preferred_element_type=jnp.float32,
    )
    h_ref[...] = jnp.maximum(ht.astype(jnp.bfloat16) + b1t_ref[...],
                             jnp.bfloat16(0.0))
    yt = jax.lax.dot_general(
        w2_ref[...], h_ref[...],
        dimension_numbers=(((0,), (0,)), ((), ())),
        preferred_element_type=jnp.float32,
    )
    tn2 = xa_ref.shape[1]
    o_ref[:, :tn2] = yt[0:_OUT_F, :]
    o_ref[:, tn2:] = yt[8:8 + _OUT_F, :]


@functools.partial(jax.jit, static_argnames=("tile_n",))
def _forward(x, params_packed, tile_n=65536):
    B = x.shape[0]
    xt = x.T
    tn = min(tile_n, max(128, -(-B // 128) * 128))
    n_pad = -(-B // (2 * tn)) * (2 * tn)
    if n_pad != B:
        xt = jnp.pad(xt, ((0, 0), (0, n_pad - B)))

    w1 = params_packed[0:_IN_F, :_HID_W]
    b1 = params_packed[8:9, :_HID_W]
    b2 = params_packed[_B2_ROW:_B2_ROW + 1, :_OUT_W]
    w2 = params_packed[_W2_ROW:_W2_ROW + _HID_W, :_OUT_W]
    b1 = b1.at[0, _HID_F].set(1.0)
    w2 = w2.at[_HID_F, :].set(b2[0, :])
    w1d = jnp.zeros((2 * _IN_F, 2 * _HID_W), jnp.bfloat16)
    w1d = w1d.at[0:_IN_F, 0:_HID_W].set(w1.astype(jnp.bfloat16))
    w1d = w1d.at[_IN_F:, _HID_W:].set(w1.astype(jnp.bfloat16))
    w2d = jnp.zeros((2 * _HID_W, 2 * _OUT_W), jnp.bfloat16)
    w2d = w2d.at[0:_HID_W, 0:_OUT_W].set(w2.astype(jnp.bfloat16))
    w2d = w2d.at[_HID_W:, _OUT_W:].set(w2.astype(jnp.bfloat16))
    b1d = jnp.concatenate([b1, b1], axis=1).T.astype(jnp.bfloat16)

    grid = (n_pad // (2 * tn),)
    cost = pl.CostEstimate(
        flops=2 * n_pad * (_IN_F * _HID_W + _HID_W * _OUT_W),
        transcendentals=0,
        bytes_accessed=(n_pad * _IN_F + 152 * 128 + n_pad * _OUT_F) * 4,
    )
    out_t = pl.pallas_call(
        _mlp_kernel_t,
        out_shape=jax.ShapeDtypeStruct((_OUT_F, n_pad), jnp.float32),
        grid=grid,
        in_specs=[
            pl.BlockSpec((_IN_F, tn), lambda i: (0, 2 * i)),
            pl.BlockSpec((_IN_F, tn), lambda i: (0, 2 * i + 1)),
            pl.BlockSpec((2 * _IN_F, 2 * _HID_W), lambda i: (0, 0)),
            pl.BlockSpec((2 * _HID_W, 1), lambda i: (0, 0)),
            pl.BlockSpec((2 * _HID_W, 2 * _OUT_W), lambda i: (0, 0)),
        ],
        out_specs=pl.BlockSpec((_OUT_F, 2 * tn), lambda i: (0, i)),
        scratch_shapes=[pltpu.VMEM((2 * _HID_W, tn), jnp.bfloat16)],
        compiler_params=pltpu.CompilerParams(
            dimension_semantics=("parallel",),
        ),
        cost_estimate=cost,
    )(xt, xt, w1d, b1d, w2d)
    return out_t[:, :B].T


def kernel(x, params_packed):
    return _forward(x, params_packed)
